# Initial kernel scaffold; baseline (speedup 1.0000x reference)
import jax
import jax.numpy as jnp
from jax.experimental import pallas as pl


def kernel(edge_index, edge_weight, entity_embed):
    raise NotImplementedError("write your pallas kernel here")


# speedup vs baseline: 2.4016x; 2.4016x over previous
import functools

import jax
import jax.numpy as jnp
from jax import lax
from jax.experimental import pallas as pl
from jax.experimental.pallas import tpu as pltpu
from jax.experimental.pallas import tpu_sc as plsc

N = 10000
NP = 10240
E = 320000
D = 128
H = 4
ITERS = 4
ALPHA = 0.15

NC = 2
NS = 16
L = 16

K = 80
EDGES_PER_TILE = E // NS
CHUNKS_PER_TILE = EDGES_PER_TILE // K
ROWS_PER_TILE = NP // NS
RCHUNK = 128
RCHUNKS = ROWS_PER_TILE // RCHUNK


def _ppr_body(col_hbm, row_hbm, w_hbm, ent_hbm, z_hbm,
              col_v, colh_v, row_v, w_v, rows_v, stage_v, acc, sem):
    c = lax.axis_index("c")
    s = lax.axis_index("s")
    r0 = s * ROWS_PER_TILE
    e0 = s * EDGES_PER_TILE

    for hh in range(H // NC):
        h = c * (H // NC) + hh
        hN = h * NP
        hE = h * E
        for it in range(ITERS):
            for rc in range(RCHUNKS):
                rbase = r0 + rc * RCHUNK
                pltpu.sync_copy(ent_hbm.at[pl.ds(rbase, RCHUNK)], stage_v)

                def _scale_row(r, _):
                    for d in range(D // L):
                        blk = stage_v[r, pl.ds(d * L, L)]
                        stage_v[r, pl.ds(d * L, L)] = blk * ALPHA
                    return _

                lax.fori_loop(0, RCHUNK, _scale_row, 0)
                pltpu.sync_copy(stage_v, acc.at[pl.ds(rbase, RCHUNK)])
            plsc.subcore_barrier()

            def _chunk(i, _):
                ebase = e0 + i * K
                pltpu.sync_copy(col_hbm.at[pl.ds(ebase, K)], col_v)
                pltpu.sync_copy(row_hbm.at[pl.ds(ebase, K)], row_v)
                pltpu.sync_copy(w_hbm.at[pl.ds(hE + ebase, K)], w_v)
                for j in range(K // L):
                    w_v[pl.ds(j * L, L)] = w_v[pl.ds(j * L, L)] * (1.0 - ALPHA)
                if it == 0:
                    pltpu.async_copy(ent_hbm.at[col_v], rows_v, sem).wait()
                else:
                    for j in range(K // L):
                        colh_v[pl.ds(j * L, L)] = col_v[pl.ds(j * L, L)] + hN
                    pltpu.async_copy(z_hbm.at[colh_v], rows_v, sem).wait()

                def _group(g, _):
                    wblk = w_v[pl.ds(g * L, L)]
                    for el in range(L):
                        wsplat = lax.gather(
                            wblk, jnp.full((L, 1), el, jnp.int32),
                            lax.GatherDimensionNumbers(
                                offset_dims=(), collapsed_slice_dims=(0,),
                                start_index_map=(0,)),
                            slice_sizes=(1,),
                            mode=lax.GatherScatterMode.PROMISE_IN_BOUNDS)
                        e = g * L + el
                        for d in range(D // L):
                            blk = rows_v[e, pl.ds(d * L, L)]
                            rows_v[e, pl.ds(d * L, L)] = blk * wsplat
                    return _

                lax.fori_loop(0, K // L, _group, 0)
                pltpu.sync_copy(rows_v, acc.at[row_v], add=True)
                return _

            lax.fori_loop(0, CHUNKS_PER_TILE, _chunk, 0)
            plsc.subcore_barrier()

            for rc in range(RCHUNKS):
                rbase = r0 + rc * RCHUNK
                pltpu.sync_copy(acc.at[pl.ds(rbase, RCHUNK)], stage_v)
                pltpu.sync_copy(stage_v, z_hbm.at[pl.ds(hN + rbase, RCHUNK)])
            plsc.subcore_barrier()


_mesh = plsc.VectorSubcoreMesh(
    core_axis_name="c", subcore_axis_name="s", num_cores=NC, num_subcores=NS)

_ppr = pl.kernel(
    _ppr_body,
    out_type=jax.ShapeDtypeStruct((H * NP, D), jnp.float32),
    mesh=_mesh,
    scratch_types=[
        pltpu.VMEM((K,), jnp.int32),
        pltpu.VMEM((K,), jnp.int32),
        pltpu.VMEM((K,), jnp.int32),
        pltpu.VMEM((K,), jnp.float32),
        pltpu.VMEM((K, D), jnp.float32),
        pltpu.VMEM((RCHUNK, D), jnp.float32),
        pltpu.VMEM_SHARED((NP, D), jnp.float32),
        pltpu.SemaphoreType.DMA,
    ],
)


@jax.jit
def kernel(edge_index, edge_weight, entity_embed):
    row = edge_index[0].astype(jnp.int32)
    col = edge_index[1].astype(jnp.int32)
    w = edge_weight.astype(jnp.float32).T.reshape(H * E)
    ent = jnp.concatenate(
        [entity_embed, jnp.zeros((NP - N, D), jnp.float32)], axis=0)
    z = _ppr(col, row, w, ent)
    return z.reshape(H, NP, D)[:, :N].transpose(1, 0, 2)


# speedup vs baseline: 2.8784x; 1.1985x over previous
import jax
import jax.numpy as jnp
from jax import lax
from jax.experimental import pallas as pl
from jax.experimental.pallas import tpu as pltpu
from jax.experimental.pallas import tpu_sc as plsc

N = 10000
NP = 10240
E = 320000
EP = 327680
D = 128
H = 4
ITERS = 4
ALPHA = 0.15

NC = 2
NS = 16
L = 16

K = 128
CHUNKS = EP // K
CPT = CHUNKS // NS
ROWS_PER_TILE = NP // NS
RCHUNK = 64
RCHUNKS = ROWS_PER_TILE // RCHUNK

R_COL = 0
R_ROW = 1


def _splat(wblk, el):
    return lax.gather(
        wblk, jnp.full((L, 1), el, jnp.int32),
        lax.GatherDimensionNumbers(
            offset_dims=(), collapsed_slice_dims=(0,), start_index_map=(0,)),
        slice_sizes=(1,),
        mode=lax.GatherScatterMode.PROMISE_IN_BOUNDS)


def _ppr_body(idx_hbm, w_hbm, ent_hbm, z_hbm,
              idx0, idx1, wb0, wb1, rows0, rows1, stage_v, acc,
              isem0, isem1, gsem0, gsem1):
    c = lax.axis_index("c")
    s = lax.axis_index("s")
    r0 = s * ROWS_PER_TILE
    c0 = s * CPT

    idxbufs = (idx0, idx1)
    wbufs = (wb0, wb1)
    rowsb = (rows0, rows1)
    isems = (isem0, isem1)
    gsems = (gsem0, gsem1)

    def _process(rows_buf, idxbuf, wbuf):
        def _group(g, _):
            wblk = wbuf[pl.ds(g * L, L)] * (1.0 - ALPHA)
            for el in range(L):
                wsplat = _splat(wblk, el)
                e = g * L + el
                for d in range(D // L):
                    blk = rows_buf[e, pl.ds(d * L, L)]
                    rows_buf[e, pl.ds(d * L, L)] = blk * wsplat
            return _

        lax.fori_loop(0, K // L, _group, 0)
        pltpu.sync_copy(rows_buf, acc.at[idxbuf.at[R_ROW]], add=True)

    def _head(hh, _):
        h = c * (H // NC) + hh
        hN = h * NP

        def _init_acc(rc, _):
            rbase = r0 + rc * RCHUNK
            pltpu.sync_copy(ent_hbm.at[pl.ds(rbase, RCHUNK)], stage_v)

            def _scale_row(r, _):
                for d in range(D // L):
                    blk = stage_v[r, pl.ds(d * L, L)]
                    stage_v[r, pl.ds(d * L, L)] = blk * ALPHA
                return _

            lax.fori_loop(0, RCHUNK, _scale_row, 0)
            pltpu.sync_copy(stage_v, acc.at[pl.ds(rbase, RCHUNK)])
            return _

        def _edge_pass(first):
            def _adjust(idxbuf):
                if not first:
                    for d in range(K // L):
                        blk = idxbuf[R_COL, pl.ds(d * L, L)]
                        idxbuf[R_COL, pl.ds(d * L, L)] = blk + hN

            def _issue_gather(idxbuf, rows_buf, gsem):
                if first:
                    pltpu.async_copy(ent_hbm.at[idxbuf.at[R_COL]], rows_buf, gsem)
                else:
                    pltpu.async_copy(z_hbm.at[idxbuf.at[R_COL]], rows_buf, gsem)

            def _issue_idx(i, b):
                pltpu.async_copy(idx_hbm.at[c0 + i], idxbufs[b], isems[b])
                pltpu.async_copy(
                    w_hbm.at[h * CHUNKS + c0 + i], wbufs[b], isems[b])

            def _wait_idx(b):
                pltpu.make_async_copy(idx_hbm.at[0], idxbufs[b], isems[b]).wait()
                pltpu.make_async_copy(w_hbm.at[0], wbufs[b], isems[b]).wait()

            def _wait_rows(buf, sem):
                pltpu.make_async_copy(ent_hbm.at[pl.ds(0, K)], buf, sem).wait()

            pltpu.sync_copy(idx_hbm.at[c0], idx0)
            pltpu.sync_copy(w_hbm.at[h * CHUNKS + c0], wb0)
            _adjust(idx0)
            _issue_gather(idx0, rows0, gsem0)
            _issue_idx(1, 1)

            def _step(i, b):
                nb = 1 - b

                @pl.when(i + 1 < CPT)
                def _stage1():
                    _wait_idx(nb)
                    _adjust(idxbufs[nb])
                    _issue_gather(idxbufs[nb], rowsb[nb], gsems[nb])

                _wait_rows(rowsb[b], gsems[b])
                _process(rowsb[b], idxbufs[b], wbufs[b])

                @pl.when(i + 2 < CPT)
                def _stage3():
                    _issue_idx(i + 2, b)

            def _pair(j, _):
                _step(2 * j, 0)
                _step(2 * j + 1, 1)
                return _

            lax.fori_loop(0, CPT // 2, _pair, 0)

        def _finish(_unused):
            plsc.subcore_barrier()

            def _wb(rc, _):
                rbase = r0 + rc * RCHUNK
                pltpu.sync_copy(acc.at[pl.ds(rbase, RCHUNK)], stage_v)
                pltpu.sync_copy(stage_v, z_hbm.at[pl.ds(hN + rbase, RCHUNK)])
                return _

            lax.fori_loop(0, RCHUNKS, _wb, 0)
            plsc.subcore_barrier()

        lax.fori_loop(0, RCHUNKS, _init_acc, 0)
        plsc.subcore_barrier()
        _edge_pass(first=True)
        _finish(0)

        def _iter(it, _):
            lax.fori_loop(0, RCHUNKS, _init_acc, 0)
            plsc.subcore_barrier()
            _edge_pass(first=False)
            _finish(0)
            return _

        lax.fori_loop(1, ITERS, _iter, 0)
        return _

    lax.fori_loop(0, H // NC, _head, 0)


_mesh = plsc.VectorSubcoreMesh(
    core_axis_name="c", subcore_axis_name="s", num_cores=NC, num_subcores=NS)

_ppr = pl.kernel(
    _ppr_body,
    out_type=jax.ShapeDtypeStruct((H * NP, D), jnp.float32),
    mesh=_mesh,
    scratch_types=[
        pltpu.VMEM((2, K), jnp.int32),
        pltpu.VMEM((2, K), jnp.int32),
        pltpu.VMEM((K,), jnp.float32),
        pltpu.VMEM((K,), jnp.float32),
        pltpu.VMEM((K, D), jnp.float32),
        pltpu.VMEM((K, D), jnp.float32),
        pltpu.VMEM((RCHUNK, D), jnp.float32),
        pltpu.VMEM_SHARED((NP, D), jnp.float32),
        pltpu.SemaphoreType.DMA,
        pltpu.SemaphoreType.DMA,
        pltpu.SemaphoreType.DMA,
        pltpu.SemaphoreType.DMA,
    ],
)


@jax.jit
def kernel(edge_index, edge_weight, entity_embed):
    row = edge_index[0].astype(jnp.int32)
    col = edge_index[1].astype(jnp.int32)
    pad = EP - E
    row = jnp.concatenate([row, jnp.zeros((pad,), jnp.int32)])
    col = jnp.concatenate([col, jnp.zeros((pad,), jnp.int32)])
    w = edge_weight.astype(jnp.float32).T
    w = jnp.concatenate([w, jnp.zeros((H, pad), jnp.float32)], axis=1)
    idx = jnp.stack([col.reshape(CHUNKS, K), row.reshape(CHUNKS, K)],
                    axis=1)
    ent = jnp.concatenate(
        [entity_embed, jnp.zeros((NP - N, D), jnp.float32)], axis=0)
    z = _ppr(idx, w.reshape(H * CHUNKS, K), ent)
    return z.reshape(H, NP, D)[:, :N].transpose(1, 0, 2)


# speedup vs baseline: 2.9595x; 1.0282x over previous
import jax
import jax.numpy as jnp
from jax import lax
from jax.experimental import pallas as pl
from jax.experimental.pallas import tpu as pltpu
from jax.experimental.pallas import tpu_sc as plsc

N = 10000
NP = 10240
E = 320000
EP = 327680
D = 128
H = 4
ITERS = 4
ALPHA = 0.15

NC = 2
NS = 16
L = 16

K = 160
KH = 80
CHUNKS = EP // K
CPT = CHUNKS // NS
ROWS_PER_TILE = NP // NS
RCHUNK = 32
RCHUNKS = ROWS_PER_TILE // RCHUNK


def _splat(wblk, el):
    return lax.gather(
        wblk, jnp.full((L, 1), el, jnp.int32),
        lax.GatherDimensionNumbers(
            offset_dims=(), collapsed_slice_dims=(0,), start_index_map=(0,)),
        slice_sizes=(1,),
        mode=lax.GatherScatterMode.PROMISE_IN_BOUNDS)


def _ppr_body(col_hbm, rowi_hbm, w_hbm, ent_hbm, z_hbm,
              cb0, cb1, rb0, rb1, wb0, wb1, rows0, rows1, stage_v, acc,
              isem0, isem1, gsem0, gsem1):
    c = lax.axis_index("c")
    s = lax.axis_index("s")
    r0 = s * ROWS_PER_TILE
    c0 = s * CPT

    colbufs = (cb0, cb1)
    rowbufs = (rb0, rb1)
    wbufs = (wb0, wb1)
    rowsb = (rows0, rows1)
    isems = (isem0, isem1)
    gsems = (gsem0, gsem1)

    def _process(rows_buf, rowbuf, wbuf):
        def _group(g, _):
            wblk = wbuf[pl.ds(g * L, L)] * (1.0 - ALPHA)
            for el in range(L):
                wsplat = _splat(wblk, el)
                e = g * L + el
                for d in range(D // L):
                    blk = rows_buf[e, pl.ds(d * L, L)]
                    rows_buf[e, pl.ds(d * L, L)] = blk * wsplat
            return _

        lax.fori_loop(0, K // L, _group, 0)
        pltpu.sync_copy(rows_buf.at[pl.ds(0, KH)],
                        acc.at[rowbuf.at[0]], add=True)
        pltpu.sync_copy(rows_buf.at[pl.ds(KH, KH)],
                        acc.at[rowbuf.at[1]], add=True)

    def _head(hh, _):
        h = c * (H // NC) + hh
        hN = h * NP

        def _init_acc(rc, _):
            rbase = r0 + rc * RCHUNK
            pltpu.sync_copy(ent_hbm.at[pl.ds(rbase, RCHUNK)], stage_v)

            def _scale_row(r, _):
                for d in range(D // L):
                    blk = stage_v[r, pl.ds(d * L, L)]
                    stage_v[r, pl.ds(d * L, L)] = blk * ALPHA
                return _

            lax.fori_loop(0, RCHUNK, _scale_row, 0)
            pltpu.sync_copy(stage_v, acc.at[pl.ds(rbase, RCHUNK)])
            return _

        def _edge_pass(first):
            def _adjust(colbuf):
                if not first:
                    for d in range(K // L):
                        blk = colbuf[pl.ds(d * L, L)]
                        colbuf[pl.ds(d * L, L)] = blk + hN

            def _issue_gather(colbuf, rows_buf, gsem):
                if first:
                    pltpu.async_copy(ent_hbm.at[colbuf], rows_buf, gsem)
                else:
                    pltpu.async_copy(z_hbm.at[colbuf], rows_buf, gsem)

            def _issue_idx(i, b):
                pltpu.async_copy(col_hbm.at[c0 + i], colbufs[b], isems[b])
                pltpu.async_copy(rowi_hbm.at[c0 + i], rowbufs[b], isems[b])
                pltpu.async_copy(
                    w_hbm.at[h * CHUNKS + c0 + i], wbufs[b], isems[b])

            def _wait_idx(b):
                pltpu.make_async_copy(col_hbm.at[0], colbufs[b], isems[b]).wait()
                pltpu.make_async_copy(rowi_hbm.at[0], rowbufs[b], isems[b]).wait()
                pltpu.make_async_copy(w_hbm.at[0], wbufs[b], isems[b]).wait()

            def _wait_rows(buf, sem):
                pltpu.make_async_copy(z_hbm.at[pl.ds(0, K)], buf, sem).wait()

            pltpu.sync_copy(col_hbm.at[c0], cb0)
            pltpu.sync_copy(rowi_hbm.at[c0], rb0)
            pltpu.sync_copy(w_hbm.at[h * CHUNKS + c0], wb0)
            _adjust(cb0)
            _issue_gather(cb0, rows0, gsem0)
            _issue_idx(1, 1)

            def _step(i, b):
                nb = 1 - b

                @pl.when(i + 1 < CPT)
                def _stage1():
                    _wait_idx(nb)
                    _adjust(colbufs[nb])
                    _issue_gather(colbufs[nb], rowsb[nb], gsems[nb])

                _wait_rows(rowsb[b], gsems[b])
                _process(rowsb[b], rowbufs[b], wbufs[b])

                @pl.when(i + 2 < CPT)
                def _stage3():
                    _issue_idx(i + 2, b)

            def _pair(j, _):
                _step(2 * j, 0)
                _step(2 * j + 1, 1)
                return _

            lax.fori_loop(0, CPT // 2, _pair, 0)

        def _finish(_unused):
            plsc.subcore_barrier()

            def _wb(rc, _):
                rbase = r0 + rc * RCHUNK
                pltpu.sync_copy(acc.at[pl.ds(rbase, RCHUNK)], stage_v)
                pltpu.sync_copy(stage_v, z_hbm.at[pl.ds(hN + rbase, RCHUNK)])
                return _

            lax.fori_loop(0, RCHUNKS, _wb, 0)
            plsc.subcore_barrier()

        lax.fori_loop(0, RCHUNKS, _init_acc, 0)
        plsc.subcore_barrier()
        _edge_pass(first=True)
        _finish(0)

        def _iter(it, _):
            lax.fori_loop(0, RCHUNKS, _init_acc, 0)
            plsc.subcore_barrier()
            _edge_pass(first=False)
            _finish(0)
            return _

        lax.fori_loop(1, ITERS, _iter, 0)
        return _

    lax.fori_loop(0, H // NC, _head, 0)


_mesh = plsc.VectorSubcoreMesh(
    core_axis_name="c", subcore_axis_name="s", num_cores=NC, num_subcores=NS)

_ppr = pl.kernel(
    _ppr_body,
    out_type=jax.ShapeDtypeStruct((H * NP, D), jnp.float32),
    mesh=_mesh,
    scratch_types=[
        pltpu.VMEM((K,), jnp.int32),
        pltpu.VMEM((K,), jnp.int32),
        pltpu.VMEM((2, KH), jnp.int32),
        pltpu.VMEM((2, KH), jnp.int32),
        pltpu.VMEM((K,), jnp.float32),
        pltpu.VMEM((K,), jnp.float32),
        pltpu.VMEM((K, D), jnp.float32),
        pltpu.VMEM((K, D), jnp.float32),
        pltpu.VMEM((RCHUNK, D), jnp.float32),
        pltpu.VMEM_SHARED((NP, D), jnp.float32),
        pltpu.SemaphoreType.DMA,
        pltpu.SemaphoreType.DMA,
        pltpu.SemaphoreType.DMA,
        pltpu.SemaphoreType.DMA,
    ],
)


@jax.jit
def kernel(edge_index, edge_weight, entity_embed):
    row = edge_index[0].astype(jnp.int32)
    col = edge_index[1].astype(jnp.int32)
    pad = EP - E
    row = jnp.concatenate([row, jnp.zeros((pad,), jnp.int32)])
    col = jnp.concatenate([col, jnp.zeros((pad,), jnp.int32)])
    w = edge_weight.astype(jnp.float32).T
    w = jnp.concatenate([w, jnp.zeros((H, pad), jnp.float32)], axis=1)
    ent = jnp.concatenate(
        [entity_embed, jnp.zeros((NP - N, D), jnp.float32)], axis=0)
    z = _ppr(col.reshape(CHUNKS, K), row.reshape(CHUNKS, 2, KH),
             w.reshape(H * CHUNKS, K), ent)
    return z.reshape(H, NP, D)[:, :N].transpose(1, 0, 2)
